# baseline (speedup 1.0000x reference)
import jax
import jax.numpy as jnp
from jax import lax
from jax.experimental import pallas as pl
from jax.experimental.pallas import tpu as pltpu
from jax.experimental.pallas import tpu_sc as plsc

D = 768
B, S = 4, 4096
N = B * S
NC, NS = 2, 16
NW = NC * NS
PER_W = N // NW
WPB = S // PER_W
C = 16
NCH = PER_W // C
NBUF = 4
LANES = 16
COLS = D // LANES


def _body(inp_ref, pos_ref, tok_tab, pos_tab, out_ref,
          idx_t, idx_p,
          tok0, tok1, tok2, tok3, pb0, pb1, pb2, pb3,
          st0, st1, st2, st3, sp0, sp1, sp2, sp3,
          sw0, sw1, sw2, sw3):
    wid = lax.axis_index("s") * NC + lax.axis_index("c")
    brow = wid // WPB
    col0 = (wid % WPB) * PER_W
    pltpu.sync_copy(inp_ref.at[brow, pl.ds(col0, PER_W)], idx_t)
    pltpu.sync_copy(pos_ref.at[brow, pl.ds(col0, PER_W)], idx_p)

    toks = (tok0, tok1, tok2, tok3)
    pbufs = (pb0, pb1, pb2, pb3)
    sts = (st0, st1, st2, st3)
    sps = (sp0, sp1, sp2, sp3)
    sws = (sw0, sw1, sw2, sw3)

    H = C // 2

    def g_descs(j, b):
        ds = []
        for h in range(2):
            ds.append(pltpu.make_async_copy(
                tok_tab.at[idx_t.at[pl.ds(j * C + h * H, H)]],
                toks[b].at[pl.ds(h * H, H)], sts[b]))
            ds.append(pltpu.make_async_copy(
                pos_tab.at[idx_p.at[pl.ds(j * C + h * H, H)]],
                pbufs[b].at[pl.ds(h * H, H)], sps[b]))
        return ds

    def g_issue(j, b):
        for d in g_descs(j, b):
            d.start()

    def g_wait(j, b):
        for d in g_descs(j, b):
            d.wait()

    def w_desc(j, b):
        return pltpu.make_async_copy(
            toks[b], out_ref.at[brow, pl.ds(col0 + j * C, C)], sws[b])

    def add(b):
        tb, pb = toks[b], pbufs[b]

        def add_row(r, _):
            for k in range(COLS):
                s = pl.ds(k * LANES, LANES)
                tb[r, s] = tb[r, s] + pb[r, s]
            return 0

        lax.fori_loop(0, C, add_row, 0)

    def step(j, b, issue_ahead=True, wait_wb=True):
        g_wait(j, b)
        if wait_wb:
            w_desc(j - 2, (b - 2) % NBUF).wait()
        if issue_ahead:
            g_issue(j + 2, (b + 2) % NBUF)
        add(b)
        w_desc(j, b).start()

    g_issue(0, 0)
    g_issue(1, 1)
    step(0, 0, wait_wb=False)
    step(1, 1, wait_wb=False)

    def mid(j2, _):
        jbase = 2 + j2 * NBUF
        for i in range(NBUF):
            step(jbase + i, (2 + i) % NBUF)
        return 0

    lax.fori_loop(0, 7, mid, 0)

    step(30, 30 % NBUF, issue_ahead=False)
    step(31, 31 % NBUF, issue_ahead=False)
    w_desc(NCH - 2, (NCH - 2) % NBUF).wait()
    w_desc(NCH - 1, (NCH - 1) % NBUF).wait()


@jax.jit
def kernel(input, pos, token_table, pos_table):
    mesh = plsc.VectorSubcoreMesh(core_axis_name="c", subcore_axis_name="s")
    k = pl.kernel(
        _body,
        mesh=mesh,
        out_type=jax.ShapeDtypeStruct((B, S, D), jnp.float32),
        scratch_types=(
            [pltpu.VMEM((PER_W,), jnp.int32)] * 2
            + [pltpu.VMEM((C, D), jnp.float32)] * (2 * NBUF)
            + [pltpu.SemaphoreType.DMA] * (3 * NBUF)
        ),
    )
    return k(input, pos, token_table, pos_table)


# speedup vs baseline: 1.0083x; 1.0083x over previous
import jax
import jax.numpy as jnp
from jax import lax
from jax.experimental import pallas as pl
from jax.experimental.pallas import tpu as pltpu
from jax.experimental.pallas import tpu_sc as plsc

D = 768
B, S = 4, 4096
N = B * S
NC, NS = 2, 16
NW = NC * NS
PER_W = N // NW
WPB = S // PER_W
C = 16
NCH = PER_W // C
NBUF = 4
LANES = 16
COLS = D // LANES


def _body(inp_ref, pos_ref, tok_tab, pos_tab, out_ref,
          idx_t, idx_p,
          tok0, tok1, tok2, tok3, pb0, pb1, pb2, pb3,
          st0, st1, st2, st3, sp0, sp1, sp2, sp3,
          sw0, sw1, sw2, sw3):
    wid = lax.axis_index("s") * NC + lax.axis_index("c")
    brow = wid // WPB
    col0 = (wid % WPB) * PER_W
    pltpu.sync_copy(inp_ref.at[brow, pl.ds(col0, PER_W)], idx_t)
    pltpu.sync_copy(pos_ref.at[brow, pl.ds(col0, PER_W)], idx_p)

    toks = (tok0, tok1, tok2, tok3)
    pbufs = (pb0, pb1, pb2, pb3)
    sts = (st0, st1, st2, st3)
    sps = (sp0, sp1, sp2, sp3)
    sws = (sw0, sw1, sw2, sw3)

    def g_descs(j, b):
        ct = pltpu.make_async_copy(
            tok_tab.at[idx_t.at[pl.ds(j * C, C)]], toks[b], sts[b])
        cp = pltpu.make_async_copy(
            pos_tab.at[idx_p.at[pl.ds(j * C, C)]], pbufs[b], sps[b])
        return ct, cp

    def g_issue(j, b):
        ct, cp = g_descs(j, b)
        ct.start()
        cp.start()

    def g_wait(j, b):
        ct, cp = g_descs(j, b)
        ct.wait()
        cp.wait()

    def w_desc(j, b):
        return pltpu.make_async_copy(
            toks[b], out_ref.at[brow, pl.ds(col0 + j * C, C)], sws[b])

    def add(b):
        tb, pb = toks[b], pbufs[b]

        def add_rows(r2, _):
            for dr in range(2):
                r = r2 * 2 + dr
                for k in range(COLS):
                    s = pl.ds(k * LANES, LANES)
                    tb[r, s] = tb[r, s] + pb[r, s]
            return 0

        lax.fori_loop(0, C // 2, add_rows, 0)

    def step(j, b, issue_ahead=True, wait_wb=True):
        g_wait(j, b)
        if wait_wb:
            w_desc(j - 2, (b - 2) % NBUF).wait()
        if issue_ahead:
            g_issue(j + 2, (b + 2) % NBUF)
        add(b)
        w_desc(j, b).start()

    g_issue(0, 0)
    g_issue(1, 1)
    step(0, 0, wait_wb=False)
    step(1, 1, wait_wb=False)

    def mid(j2, _):
        jbase = 2 + j2 * NBUF
        for i in range(NBUF):
            step(jbase + i, (2 + i) % NBUF)
        return 0

    lax.fori_loop(0, 7, mid, 0)

    step(30, 30 % NBUF, issue_ahead=False)
    step(31, 31 % NBUF, issue_ahead=False)
    w_desc(NCH - 2, (NCH - 2) % NBUF).wait()
    w_desc(NCH - 1, (NCH - 1) % NBUF).wait()


@jax.jit
def kernel(input, pos, token_table, pos_table):
    mesh = plsc.VectorSubcoreMesh(core_axis_name="c", subcore_axis_name="s")
    k = pl.kernel(
        _body,
        mesh=mesh,
        out_type=jax.ShapeDtypeStruct((B, S, D), jnp.float32),
        scratch_types=(
            [pltpu.VMEM((PER_W,), jnp.int32)] * 2
            + [pltpu.VMEM((C, D), jnp.float32)] * (2 * NBUF)
            + [pltpu.SemaphoreType.DMA] * (3 * NBUF)
        ),
    )
    return k(input, pos, token_table, pos_table)
